# baseline (speedup 1.0000x reference)
import jax
import jax.numpy as jnp
from jax import lax
from jax.experimental import pallas as pl
from jax.experimental.pallas import tpu as pltpu
from jax.experimental.pallas import tpu_sc as plsc

QSIZE = 65536
DIM = 128
B = 1024
NW = 32
STRIPE = QSIZE // NW


def _sc_body(queue_hbm, key_hbm, ptr_hbm, out_hbm, ptr_v, sems):
    wid = lax.axis_index("s") * 2 + lax.axis_index("c")
    base = pl.multiple_of(wid * STRIPE, STRIPE)
    pltpu.sync_copy(ptr_hbm, ptr_v)
    p = ptr_v[...][0]

    copy = pltpu.make_async_copy(
        queue_hbm.at[pl.ds(base, STRIPE)],
        out_hbm.at[pl.ds(base, STRIPE)],
        sems.at[0],
    )
    copy.start()
    copy.wait()

    is_owner = jnp.logical_and(p >= base, p < base + STRIPE)

    @pl.when(is_owner)
    def _():
        kc = pltpu.make_async_copy(
            key_hbm,
            out_hbm.at[pl.ds(pl.multiple_of(p, B), B)],
            sems.at[1],
        )
        kc.start()
        kc.wait()


def kernel(queue, key_batch, queue_ptr):
    ptr = jnp.full((16,), queue_ptr, jnp.int32)
    mesh = plsc.VectorSubcoreMesh(core_axis_name="c", subcore_axis_name="s")
    f = pl.kernel(
        _sc_body,
        out_type=jax.ShapeDtypeStruct((QSIZE, DIM), jnp.float32),
        mesh=mesh,
        scratch_types=[
            pltpu.VMEM((16,), jnp.int32),
            pltpu.SemaphoreType.DMA((2,)),
        ],
    )
    return f(queue, key_batch, ptr)


# speedup vs baseline: 13.5387x; 13.5387x over previous
import jax
import jax.numpy as jnp
from jax.experimental import pallas as pl
from jax.experimental.pallas import tpu as pltpu

QSIZE = 65536
DIM = 128
B = 1024
CH = 1024
NCHUNK = QSIZE // CH
NBUF = 8


def _body(ptr_ref, q_hbm, k_hbm, o_hbm, bufs, sems):
    p = ptr_ref[0]
    pblk = p // CH

    def start_in(k):
        buf = bufs.at[k % NBUF]
        sem = sems.at[k % NBUF]

        @pl.when(k == pblk)
        def _():
            pltpu.make_async_copy(k_hbm, buf, sem).start()

        @pl.when(k != pblk)
        def _():
            pltpu.make_async_copy(q_hbm.at[pl.ds(k * CH, CH)], buf, sem).start()

    def wait_in(k):
        pltpu.make_async_copy(
            q_hbm.at[pl.ds(k * CH, CH)], bufs.at[k % NBUF], sems.at[k % NBUF]
        ).wait()

    def out_copy(k):
        return pltpu.make_async_copy(
            bufs.at[k % NBUF],
            o_hbm.at[pl.ds(k * CH, CH)],
            sems.at[NBUF + k % NBUF],
        )

    for j in range(NBUF):
        start_in(j)
    for k in range(NCHUNK):
        if k >= NBUF:
            out_copy(k - NBUF).wait()
            start_in(k)
        wait_in(k)
        out_copy(k).start()
    for k in range(NCHUNK - NBUF, NCHUNK):
        out_copy(k).wait()


def kernel(queue, key_batch, queue_ptr):
    ptr = jnp.asarray(queue_ptr, jnp.int32).reshape((1,))
    return pl.pallas_call(
        _body,
        out_shape=jax.ShapeDtypeStruct((QSIZE, DIM), jnp.float32),
        in_specs=[
            pl.BlockSpec(memory_space=pltpu.SMEM),
            pl.BlockSpec(memory_space=pl.ANY),
            pl.BlockSpec(memory_space=pl.ANY),
        ],
        out_specs=pl.BlockSpec(memory_space=pl.ANY),
        scratch_shapes=[
            pltpu.VMEM((NBUF, CH, DIM), jnp.float32),
            pltpu.SemaphoreType.DMA((2 * NBUF,)),
        ],
    )(ptr, queue, key_batch)


# speedup vs baseline: 28.6761x; 2.1181x over previous
import jax
import jax.numpy as jnp
from jax.experimental import pallas as pl
from jax.experimental.pallas import tpu as pltpu

QSIZE = 65536
DIM = 128
B = 1024
CH = 4096
NCHUNK = QSIZE // CH
NBUF = 4


def _body(ptr_ref, q_hbm, k_hbm, o_hbm, bufs, sems):
    p = ptr_ref[0]
    pblk = p // CH

    def start_in(k):
        buf = bufs.at[k % NBUF]
        sem = sems.at[k % NBUF]
        pltpu.make_async_copy(q_hbm.at[pl.ds(k * CH, CH)], buf, sem).start()

    def wait_in(k):
        pltpu.make_async_copy(
            q_hbm.at[pl.ds(k * CH, CH)], bufs.at[k % NBUF], sems.at[k % NBUF]
        ).wait()

    def out_copy(k):
        return pltpu.make_async_copy(
            bufs.at[k % NBUF],
            o_hbm.at[pl.ds(k * CH, CH)],
            sems.at[NBUF + k % NBUF],
        )

    for j in range(NBUF):
        start_in(j)
    for k in range(NCHUNK):
        if k >= NBUF:
            out_copy(k - NBUF).wait()
            start_in(k)
        wait_in(k)

        @pl.when(k == pblk)
        def _():
            kc = pltpu.make_async_copy(
                k_hbm,
                bufs.at[k % NBUF].at[pl.ds(p - k * CH, B)],
                sems.at[NBUF + k % NBUF],
            )
            kc.start()
            kc.wait()

        out_copy(k).start()
    for k in range(NCHUNK - NBUF, NCHUNK):
        out_copy(k).wait()


def kernel(queue, key_batch, queue_ptr):
    ptr = jnp.asarray(queue_ptr, jnp.int32).reshape((1,))
    return pl.pallas_call(
        _body,
        out_shape=jax.ShapeDtypeStruct((QSIZE, DIM), jnp.float32),
        in_specs=[
            pl.BlockSpec(memory_space=pltpu.SMEM),
            pl.BlockSpec(memory_space=pl.ANY),
            pl.BlockSpec(memory_space=pl.ANY),
        ],
        out_specs=pl.BlockSpec(memory_space=pl.ANY),
        scratch_shapes=[
            pltpu.VMEM((NBUF, CH, DIM), jnp.float32),
            pltpu.SemaphoreType.DMA((2 * NBUF,)),
        ],
    )(ptr, queue, key_batch)


# speedup vs baseline: 47.6720x; 1.6624x over previous
import jax
import jax.numpy as jnp
from jax.experimental import pallas as pl
from jax.experimental.pallas import tpu as pltpu

QSIZE = 65536
DIM = 128
B = 1024
NBLK = 4
BLK = QSIZE // NBLK
HALF = BLK // 2


def _body(ptr_ref, qa_ref, qb_ref, k_ref, o_ref):
    i = pl.program_id(0)
    p = ptr_ref[0]
    o_ref[pl.ds(0, HALF), :] = qa_ref[...]
    o_ref[pl.ds(HALF, HALF), :] = qb_ref[...]

    @pl.when(i == p // BLK)
    def _():
        o_ref[pl.ds(p % BLK, B), :] = k_ref[...]


def kernel(queue, key_batch, queue_ptr):
    ptr = jnp.asarray(queue_ptr, jnp.int32).reshape((1,))
    return pl.pallas_call(
        _body,
        grid=(NBLK,),
        out_shape=jax.ShapeDtypeStruct((QSIZE, DIM), jnp.float32),
        in_specs=[
            pl.BlockSpec(memory_space=pltpu.SMEM),
            pl.BlockSpec((HALF, DIM), lambda i: (2 * i, 0)),
            pl.BlockSpec((HALF, DIM), lambda i: (2 * i + 1, 0)),
            pl.BlockSpec((B, DIM), lambda i: (0, 0)),
        ],
        out_specs=pl.BlockSpec((BLK, DIM), lambda i: (i, 0)),
    )(ptr, queue, queue, key_batch)


# speedup vs baseline: 47.8608x; 1.0040x over previous
import jax
import jax.numpy as jnp
from jax.experimental import pallas as pl
from jax.experimental.pallas import tpu as pltpu

QSIZE = 65536
DIM = 128
B = 1024
NBLK = 4
BLK = QSIZE // NBLK


def _body(ptr_ref, q_ref, k_ref, o_ref):
    i = pl.program_id(0)
    p = ptr_ref[0]
    o_ref[...] = q_ref[...]

    @pl.when(i == p // BLK)
    def _():
        o_ref[pl.ds(p % BLK, B), :] = k_ref[...]


def kernel(queue, key_batch, queue_ptr):
    ptr = jnp.asarray(queue_ptr, jnp.int32).reshape((1,))
    return pl.pallas_call(
        _body,
        grid=(NBLK,),
        out_shape=jax.ShapeDtypeStruct((QSIZE, DIM), jnp.float32),
        in_specs=[
            pl.BlockSpec(memory_space=pltpu.SMEM),
            pl.BlockSpec((BLK, DIM), lambda i: (i, 0)),
            pl.BlockSpec((B, DIM), lambda i: (0, 0)),
        ],
        out_specs=pl.BlockSpec((BLK, DIM), lambda i: (i, 0)),
    )(ptr, queue, key_batch)
